# baseline (speedup 1.0000x reference)
import functools

import jax
import jax.numpy as jnp
from jax import lax
from jax.experimental import pallas as pl
from jax.experimental.pallas import tpu as pltpu
from jax.experimental.pallas import tpu_sc as plsc

N = 10000
D = 128
EPS = 1e-5

N_PAD = 10112
E_PAD = 327680
ROWS_PER_TILE = 632
CHUNK = 128
NCH_A = 108
NCH_B = 52


def _sc_mesh():
    return plsc.VectorSubcoreMesh(core_axis_name="c", subcore_axis_name="s")


def _deg_hist_call(dst2d):
    blk = 64

    @functools.partial(
        pl.kernel,
        mesh=_sc_mesh(),
        out_type=jax.ShapeDtypeStruct((2 * N_PAD * 16,), jnp.float32),
        scratch_types=[
            pltpu.VMEM((blk, CHUNK), jnp.int32),
            pltpu.VMEM((ROWS_PER_TILE * 16,), jnp.float32),
        ],
        compiler_params=pltpu.CompilerParams(needs_layout_passes=False),
    )
    def k(dst_hbm, out_hbm, dbuf, hist):
        c = lax.axis_index("c")
        s = lax.axis_index("s")
        lo = s * ROWS_PER_TILE
        lanes = lax.broadcasted_iota(jnp.int32, (16,), 0)
        ones16 = jnp.ones((16,), jnp.float32)

        def zrow(r, carry):
            hist[pl.ds(r * 16, 16)] = jnp.zeros((16,), jnp.float32)
            return carry

        lax.fori_loop(0, ROWS_PER_TILE, zrow, 0)

        half_rows = E_PAD // CHUNK // 2
        n_blk = half_rows // blk
        row_base = c * half_rows

        def brow(r, carry):
            for g in range(CHUNK // 16):
                d = dbuf[r, pl.ds(g * 16, 16)]
                loc = jnp.clip(d - lo, 0, ROWS_PER_TILE - 1)
                m = (d >= lo) & (d < lo + ROWS_PER_TILE)
                plsc.addupdate_scatter(hist, [loc * 16 + lanes], ones16,
                                       mask=m)
            return carry

        def bblk(i, carry):
            pltpu.sync_copy(dst_hbm.at[pl.ds(row_base + i * blk, blk)], dbuf)
            lax.fori_loop(0, blk, brow, 0)
            return carry

        lax.fori_loop(0, n_blk, bblk, 0)
        pltpu.sync_copy(
            hist, out_hbm.at[pl.ds((c * N_PAD + lo) * 16, ROWS_PER_TILE * 16)])

    return k(dst2d).reshape(2 * N_PAD, 16)


def _edge_scatter_call(table, src_p, dst_p, zeros, width):
    @functools.partial(
        pl.kernel,
        mesh=_sc_mesh(),
        out_type=jax.ShapeDtypeStruct((2 * N_PAD, width), jnp.float32),
        scratch_types=[
            pltpu.VMEM((CHUNK,), jnp.int32),
            pltpu.VMEM((CHUNK,), jnp.int32),
            pltpu.VMEM((CHUNK,), jnp.int32),
            pltpu.VMEM((CHUNK,), jnp.int32),
            pltpu.VMEM((CHUNK, width), jnp.float32),
            pltpu.VMEM((CHUNK, width), jnp.float32),
            pltpu.SemaphoreType.DMA,
            pltpu.VMEM_SHARED((N_PAD, width), jnp.float32),
        ],
    )
    def k(table_hbm, src_hbm, dst_hbm, zeros_hbm, out_hbm,
          sidx0, didx0, sidx1, didx1, rows0, rows1, sem, agg_sh):
        c = lax.axis_index("c")
        s = lax.axis_index("s")
        row0 = s * ROWS_PER_TILE
        pltpu.sync_copy(zeros_hbm.at[pl.ds(row0, ROWS_PER_TILE)],
                        agg_sh.at[pl.ds(row0, ROWS_PER_TILE)])
        plsc.subcore_barrier()
        n_my = jnp.where(c == 0, NCH_A, NCH_B)
        ebase = jnp.where(c == 0, s * NCH_A, 16 * NCH_A + s * NCH_B) * CHUNK
        sidx = (sidx0, sidx1)
        didx = (didx0, didx1)
        rows = (rows0, rows1)

        pltpu.sync_copy(src_hbm.at[pl.ds(ebase, CHUNK)], sidx0)
        pltpu.sync_copy(dst_hbm.at[pl.ds(ebase, CHUNK)], didx0)
        pltpu.async_copy(table_hbm.at[sidx0], rows0, sem).wait()

        def body2(i, b, nb):
            e0 = ebase + (i + 1) * CHUNK
            pltpu.sync_copy(src_hbm.at[pl.ds(e0, CHUNK)], sidx[nb])
            pltpu.sync_copy(dst_hbm.at[pl.ds(e0, CHUNK)], didx[nb])
            cp = pltpu.async_copy(table_hbm.at[sidx[nb]], rows[nb], sem)
            pltpu.sync_copy(rows[b], agg_sh.at[didx[b]], add=True)
            cp.wait()

        def body(i2, carry):
            i = i2 * 2
            body2(i, 0, 1)
            body2(i + 1, 1, 0)
            return carry

        lax.fori_loop(0, (n_my - 2) // 2, body, 0)
        body2(n_my - 2, 0, 1)
        pltpu.sync_copy(rows1, agg_sh.at[didx1], add=True)

        plsc.subcore_barrier()
        pltpu.sync_copy(agg_sh.at[pl.ds(row0, ROWS_PER_TILE)],
                        out_hbm.at[pl.ds(c * N_PAD + row0, ROWS_PER_TILE)])

    return k(table, src_p, dst_p, zeros)


def _proj_bn_call(x, W_proj, b_proj, gamma, beta, degp):
    def body(x_ref, wp_ref, bp_ref, g_ref, bt_ref, degp_ref, hs0_ref, isd_ref):
        h = jnp.dot(x_ref[...], wp_ref[...],
                    preferred_element_type=jnp.float32) + bp_ref[...]
        mean = jnp.mean(h, axis=0, keepdims=True)
        ctr = h - mean
        var = jnp.mean(ctr * ctr, axis=0, keepdims=True)
        hbn = ctr * lax.rsqrt(var + EPS) * g_ref[...] + bt_ref[...]
        deg = jnp.sum(degp_ref[0:N_PAD, :] + degp_ref[N_PAD:2 * N_PAD, :],
                      axis=1, keepdims=True) + 1.0
        isd = lax.rsqrt(deg)
        rowid = lax.broadcasted_iota(jnp.int32, (N_PAD, 1), 0)
        isd = jnp.where(rowid < N, isd, 0.0)
        isd_b = jnp.broadcast_to(isd, (N_PAD, D))
        isd_ref[...] = isd_b
        hs0_ref[0:N, :] = isd_b[0:N, :] * hbn
        hs0_ref[N:N_PAD, :] = jnp.zeros((N_PAD - N, D), jnp.float32)

    return pl.pallas_call(
        body,
        out_shape=(jax.ShapeDtypeStruct((N_PAD, D), jnp.float32),
                   jax.ShapeDtypeStruct((N_PAD, D), jnp.float32)),
    )(x, W_proj, b_proj, gamma, beta, degp)


def _layer_call(aggp, hs_prev, isd_b, W, b):
    def body(aggp_ref, hs_ref, isd_ref, w_ref, b_ref, out_ref):
        isd = isd_ref[...]
        full = isd * (aggp_ref[0:N_PAD, :] + aggp_ref[N_PAD:2 * N_PAD, :]
                      + hs_ref[...])
        h = jnp.maximum(jnp.dot(full, w_ref[...],
                                preferred_element_type=jnp.float32)
                        + b_ref[...], 0.0)
        out_ref[...] = isd * h

    return pl.pallas_call(
        body,
        out_shape=jax.ShapeDtypeStruct((N_PAD, D), jnp.float32),
    )(aggp, hs_prev, isd_b, W, b)


def _final_call(aggp, hs_prev, isd_b, W1, b1, W_out, b_out):
    def body(aggp_ref, hs_ref, isd_ref, w1_ref, b1_ref, wo_ref, bo_ref,
             out_ref):
        isd = isd_ref[...]
        full = isd * (aggp_ref[0:N_PAD, :] + aggp_ref[N_PAD:2 * N_PAD, :]
                      + hs_ref[...])
        h = jnp.maximum(jnp.dot(full, w1_ref[...],
                                preferred_element_type=jnp.float32)
                        + b1_ref[...], 0.0)
        out = jnp.dot(h[0:N, :], wo_ref[...],
                      preferred_element_type=jnp.float32) + bo_ref[...]
        out_ref[...] = out

    return pl.pallas_call(
        body,
        out_shape=jax.ShapeDtypeStruct((N, 3), jnp.float32),
    )(aggp, hs_prev, isd_b, W1, b1, W_out, b_out)


def kernel(x, edge_index, W_proj, b_proj, gamma, beta, W0, b0, W1, b1,
           W_out, b_out):
    e = edge_index.shape[1]
    pad = jnp.full((E_PAD - e,), N, dtype=edge_index.dtype)
    src_p = jnp.concatenate([edge_index[0], pad])
    dst_p = jnp.concatenate([edge_index[1], pad])
    zeros = jnp.zeros((N_PAD, D), jnp.float32)

    degp = _deg_hist_call(dst_p.reshape(E_PAD // CHUNK, CHUNK))
    hs0, isd_b = _proj_bn_call(x, W_proj, b_proj.reshape(1, D),
                               gamma.reshape(1, D), beta.reshape(1, D), degp)
    agg1 = _edge_scatter_call(hs0, src_p, dst_p, zeros, D)
    hs1 = _layer_call(agg1, hs0, isd_b, W0, b0.reshape(1, D))
    agg2 = _edge_scatter_call(hs1, src_p, dst_p, zeros, D)
    out = _final_call(agg2, hs1, isd_b, W1, b1.reshape(1, D),
                      W_out, b_out.reshape(1, 3))
    return out


# speedup vs baseline: 1.0912x; 1.0912x over previous
import functools

import jax
import jax.numpy as jnp
from jax import lax
from jax.experimental import pallas as pl
from jax.experimental.pallas import tpu as pltpu
from jax.experimental.pallas import tpu_sc as plsc

N = 10000
D = 128
EPS = 1e-5

N_PAD = 10112
E_PAD = 327680
ROWS_PER_TILE = 632
CHUNK = 128
NCH_A = 120
NCH_B = 40


def _sc_mesh():
    return plsc.VectorSubcoreMesh(core_axis_name="c", subcore_axis_name="s")


def _deg_hist_call(dst2d):
    blk = 64

    @functools.partial(
        pl.kernel,
        mesh=_sc_mesh(),
        out_type=jax.ShapeDtypeStruct((2 * N_PAD * 16,), jnp.float32),
        scratch_types=[
            pltpu.VMEM((blk, CHUNK), jnp.int32),
            pltpu.VMEM((ROWS_PER_TILE * 16,), jnp.float32),
        ],
        compiler_params=pltpu.CompilerParams(needs_layout_passes=False),
    )
    def k(dst_hbm, out_hbm, dbuf, hist):
        c = lax.axis_index("c")
        s = lax.axis_index("s")
        lo = s * ROWS_PER_TILE
        lanes = lax.broadcasted_iota(jnp.int32, (16,), 0)
        ones16 = jnp.ones((16,), jnp.float32)

        def zrow(r, carry):
            hist[pl.ds(r * 16, 16)] = jnp.zeros((16,), jnp.float32)
            return carry

        lax.fori_loop(0, ROWS_PER_TILE, zrow, 0)

        half_rows = E_PAD // CHUNK // 2
        n_blk = half_rows // blk
        row_base = c * half_rows

        def brow(r, carry):
            for g in range(CHUNK // 16):
                d = dbuf[r, pl.ds(g * 16, 16)]
                loc = jnp.clip(d - lo, 0, ROWS_PER_TILE - 1)
                m = (d >= lo) & (d < lo + ROWS_PER_TILE)
                plsc.addupdate_scatter(hist, [loc * 16 + lanes], ones16,
                                       mask=m)
            return carry

        def bblk(i, carry):
            pltpu.sync_copy(dst_hbm.at[pl.ds(row_base + i * blk, blk)], dbuf)
            lax.fori_loop(0, blk, brow, 0)
            return carry

        lax.fori_loop(0, n_blk, bblk, 0)
        pltpu.sync_copy(
            hist, out_hbm.at[pl.ds((c * N_PAD + lo) * 16, ROWS_PER_TILE * 16)])

    return k(dst2d).reshape(2 * N_PAD, 16)


def _edge_scatter_call(table, src_p, dst_p, zeros, width):
    @functools.partial(
        pl.kernel,
        mesh=_sc_mesh(),
        out_type=jax.ShapeDtypeStruct((2 * N_PAD, width), jnp.float32),
        scratch_types=[
            pltpu.VMEM((CHUNK,), jnp.int32),
            pltpu.VMEM((CHUNK,), jnp.int32),
            pltpu.VMEM((CHUNK,), jnp.int32),
            pltpu.VMEM((CHUNK,), jnp.int32),
            pltpu.VMEM((CHUNK, width), jnp.float32),
            pltpu.VMEM((CHUNK, width), jnp.float32),
            pltpu.SemaphoreType.DMA,
            pltpu.SemaphoreType.DMA,
            pltpu.SemaphoreType.DMA,
            pltpu.VMEM_SHARED((N_PAD, width), jnp.float32),
        ],
    )
    def k(table_hbm, src_hbm, dst_hbm, zeros_hbm, out_hbm,
          sidx0, didx0, sidx1, didx1, rows0, rows1, g0, g1, ssem, agg_sh):
        c = lax.axis_index("c")
        s = lax.axis_index("s")
        row0 = s * ROWS_PER_TILE
        pltpu.sync_copy(zeros_hbm.at[pl.ds(row0, ROWS_PER_TILE)],
                        agg_sh.at[pl.ds(row0, ROWS_PER_TILE)])
        plsc.subcore_barrier()
        n_my = jnp.where(c == 0, NCH_A, NCH_B)
        ebase = jnp.where(c == 0, s * NCH_A, 16 * NCH_A + s * NCH_B) * CHUNK
        sidx = (sidx0, sidx1)
        didx = (didx0, didx1)
        rows = (rows0, rows1)
        gsem = (g0, g1)

        def load_idx(i, b):
            e0 = ebase + i * CHUNK
            pltpu.sync_copy(src_hbm.at[pl.ds(e0, CHUNK)], sidx[b])
            pltpu.sync_copy(dst_hbm.at[pl.ds(e0, CHUNK)], didx[b])

        def gather(b):
            pltpu.async_copy(table_hbm.at[sidx[b]], rows[b], gsem[b])

        def drain_gather(b):
            pltpu.make_async_copy(table_hbm.at[sidx[b]], rows[b],
                                  gsem[b]).wait()

        def scatter(b):
            pltpu.async_copy(rows[b], agg_sh.at[didx[b]], ssem, add=True)

        def drain_scatter(b):
            pltpu.make_async_copy(rows[b], agg_sh.at[didx[b]], ssem).wait()

        load_idx(0, 0)
        gather(0)
        load_idx(1, 1)
        gather(1)
        drain_gather(0)
        scatter(0)

        def step(i, b, nb):
            drain_scatter(nb)
            load_idx(i + 1, nb)
            gather(nb)
            drain_gather(b)
            scatter(b)

        def body(i2, carry):
            i = 1 + i2 * 2
            step(i, 1, 0)
            step(i + 1, 0, 1)
            return carry

        lax.fori_loop(0, (n_my - 2) // 2, body, 0)
        drain_scatter(0)
        drain_gather(1)
        scatter(1)
        drain_scatter(1)

        plsc.subcore_barrier()
        pltpu.sync_copy(agg_sh.at[pl.ds(row0, ROWS_PER_TILE)],
                        out_hbm.at[pl.ds(c * N_PAD + row0, ROWS_PER_TILE)])

    return k(table, src_p, dst_p, zeros)


def _proj_bn_call(x, W_proj, b_proj, gamma, beta, degp):
    def body(x_ref, wp_ref, bp_ref, g_ref, bt_ref, degp_ref, hs0_ref, isd_ref):
        h = jnp.dot(x_ref[...], wp_ref[...],
                    preferred_element_type=jnp.float32) + bp_ref[...]
        mean = jnp.mean(h, axis=0, keepdims=True)
        ctr = h - mean
        var = jnp.mean(ctr * ctr, axis=0, keepdims=True)
        hbn = ctr * lax.rsqrt(var + EPS) * g_ref[...] + bt_ref[...]
        deg = jnp.sum(degp_ref[0:N_PAD, :] + degp_ref[N_PAD:2 * N_PAD, :],
                      axis=1, keepdims=True) + 1.0
        isd = lax.rsqrt(deg)
        rowid = lax.broadcasted_iota(jnp.int32, (N_PAD, 1), 0)
        isd = jnp.where(rowid < N, isd, 0.0)
        isd_b = jnp.broadcast_to(isd, (N_PAD, D))
        isd_ref[...] = isd_b
        hs0_ref[0:N, :] = isd_b[0:N, :] * hbn
        hs0_ref[N:N_PAD, :] = jnp.zeros((N_PAD - N, D), jnp.float32)

    return pl.pallas_call(
        body,
        out_shape=(jax.ShapeDtypeStruct((N_PAD, D), jnp.float32),
                   jax.ShapeDtypeStruct((N_PAD, D), jnp.float32)),
    )(x, W_proj, b_proj, gamma, beta, degp)


def _layer_call(aggp, hs_prev, isd_b, W, b):
    def body(aggp_ref, hs_ref, isd_ref, w_ref, b_ref, out_ref):
        isd = isd_ref[...]
        full = isd * (aggp_ref[0:N_PAD, :] + aggp_ref[N_PAD:2 * N_PAD, :]
                      + hs_ref[...])
        h = jnp.maximum(jnp.dot(full, w_ref[...],
                                preferred_element_type=jnp.float32)
                        + b_ref[...], 0.0)
        out_ref[...] = isd * h

    return pl.pallas_call(
        body,
        out_shape=jax.ShapeDtypeStruct((N_PAD, D), jnp.float32),
    )(aggp, hs_prev, isd_b, W, b)


def _final_call(aggp, hs_prev, isd_b, W1, b1, W_out, b_out):
    def body(aggp_ref, hs_ref, isd_ref, w1_ref, b1_ref, wo_ref, bo_ref,
             out_ref):
        isd = isd_ref[...]
        full = isd * (aggp_ref[0:N_PAD, :] + aggp_ref[N_PAD:2 * N_PAD, :]
                      + hs_ref[...])
        h = jnp.maximum(jnp.dot(full, w1_ref[...],
                                preferred_element_type=jnp.float32)
                        + b1_ref[...], 0.0)
        out = jnp.dot(h[0:N, :], wo_ref[...],
                      preferred_element_type=jnp.float32) + bo_ref[...]
        out_ref[...] = out

    return pl.pallas_call(
        body,
        out_shape=jax.ShapeDtypeStruct((N, 3), jnp.float32),
    )(aggp, hs_prev, isd_b, W1, b1, W_out, b_out)


def kernel(x, edge_index, W_proj, b_proj, gamma, beta, W0, b0, W1, b1,
           W_out, b_out):
    e = edge_index.shape[1]
    pad = jnp.full((E_PAD - e,), N, dtype=edge_index.dtype)
    src_p = jnp.concatenate([edge_index[0], pad])
    dst_p = jnp.concatenate([edge_index[1], pad])
    zeros = jnp.zeros((N_PAD, D), jnp.float32)

    degp = _deg_hist_call(dst_p.reshape(E_PAD // CHUNK, CHUNK))
    hs0, isd_b = _proj_bn_call(x, W_proj, b_proj.reshape(1, D),
                               gamma.reshape(1, D), beta.reshape(1, D), degp)
    agg1 = _edge_scatter_call(hs0, src_p, dst_p, zeros, D)
    hs1 = _layer_call(agg1, hs0, isd_b, W0, b0.reshape(1, D))
    agg2 = _edge_scatter_call(hs1, src_p, dst_p, zeros, D)
    out = _final_call(agg2, hs1, isd_b, W1, b1.reshape(1, D),
                      W_out, b_out.reshape(1, 3))
    return out
